# Initial kernel scaffold; baseline (speedup 1.0000x reference)
import functools

import jax
import jax.numpy as jnp
from jax import lax
from jax.experimental import pallas as pl
from jax.experimental.pallas import tpu as pltpu
from jax.experimental.pallas import tpu_sc as plsc

VOCAB = 100000
DIM = 64
B = 16384
L = 50

NC = 2
NS = 16
NW = NC * NS
ROWS = B * L
RPW = ROWS // NW
CH = 128
NCHUNK = RPW // CH
NBUF = 4


def _make_kernel():
  mesh = plsc.VectorSubcoreMesh(core_axis_name="c", subcore_axis_name="s")

  @functools.partial(
      pl.kernel,
      mesh=mesh,
      out_type=jax.ShapeDtypeStruct((ROWS, DIM), jnp.float32),
      scratch_types=[
          pltpu.VMEM((NCHUNK, CH), jnp.int32),
          pltpu.VMEM((CH, DIM), jnp.float32),
          pltpu.VMEM((CH, DIM), jnp.float32),
          pltpu.VMEM((CH, DIM), jnp.float32),
          pltpu.VMEM((CH, DIM), jnp.float32),
          pltpu.SemaphoreType.DMA,
          pltpu.SemaphoreType.DMA,
      ],
  )
  def emb(table_hbm, idx_hbm, out_hbm, idx_v, b0, b1, b2, b3, gsem, wsem):
    bufs = (b0, b1, b2, b3)
    wid = lax.axis_index("s") * NC + lax.axis_index("c")
    base = wid * RPW

    pltpu.sync_copy(idx_hbm.at[wid], idx_v)

    for b in range(NBUF):
      pltpu.async_copy(table_hbm.at[idx_v.at[b]], bufs[b], gsem)

    def body(g, carry):
      for b in range(NBUF):
        j = g * NBUF + b
        buf = bufs[b]
        pltpu.make_async_copy(table_hbm.at[idx_v.at[j]], buf, gsem).wait()
        dst = out_hbm.at[pl.ds(base + j * CH, CH)]
        pltpu.async_copy(buf, dst, wsem)

        @pl.when(j + NBUF < NCHUNK)
        def _():
          pltpu.make_async_copy(buf, dst, wsem).wait()
          pltpu.async_copy(table_hbm.at[idx_v.at[j + NBUF]], buf, gsem)

      return carry

    lax.fori_loop(0, NCHUNK // NBUF, body, 0)

    for b in range(NBUF):
      pltpu.make_async_copy(
          bufs[b], out_hbm.at[pl.ds(base, CH)], wsem).wait()

  return emb


_emb = _make_kernel()


@jax.jit
def kernel(x, table):
  idx = x.reshape(NW, NCHUNK, CH).astype(jnp.int32)
  out = _emb(table, idx)
  return out.reshape(B, L, DIM)


# speedup vs baseline: 6.2540x; 6.2540x over previous
import functools

import jax
import jax.numpy as jnp
from jax import lax
from jax.experimental import pallas as pl
from jax.experimental.pallas import tpu as pltpu
from jax.experimental.pallas import tpu_sc as plsc

VOCAB = 100000
DIM = 64
B = 16384
L = 50

NC = 2
NS = 16
NW = NC * NS
ROWS = B * L
RPW = ROWS // NW
CH = 128
NCHUNK = RPW // CH
NBUF = 4


def _make_kernel():
  mesh = plsc.VectorSubcoreMesh(core_axis_name="c", subcore_axis_name="s")

  @functools.partial(
      pl.kernel,
      mesh=mesh,
      compiler_params=pltpu.CompilerParams(use_tc_tiling_on_sc=False),
      out_type=jax.ShapeDtypeStruct((ROWS, DIM), jnp.float32),
      scratch_types=[
          pltpu.VMEM((NCHUNK, CH), jnp.int32),
          pltpu.VMEM((CH, DIM), jnp.float32),
          pltpu.VMEM((CH, DIM), jnp.float32),
          pltpu.VMEM((CH, DIM), jnp.float32),
          pltpu.VMEM((CH, DIM), jnp.float32),
          pltpu.SemaphoreType.DMA,
          pltpu.SemaphoreType.DMA,
      ],
  )
  def emb(table_hbm, idx_hbm, out_hbm, idx_v, b0, b1, b2, b3, gsem, wsem):
    bufs = (b0, b1, b2, b3)
    wid = lax.axis_index("s") * NC + lax.axis_index("c")
    base = wid * RPW

    pltpu.sync_copy(idx_hbm.at[wid], idx_v)

    for b in range(NBUF):
      pltpu.async_copy(table_hbm.at[idx_v.at[b]], bufs[b], gsem)

    def body(g, carry):
      for b in range(NBUF):
        j = g * NBUF + b
        buf = bufs[b]
        pltpu.make_async_copy(table_hbm.at[idx_v.at[j]], buf, gsem).wait()
        dst = out_hbm.at[pl.ds(base + j * CH, CH)]
        pltpu.async_copy(buf, dst, wsem)

        @pl.when(j + NBUF < NCHUNK)
        def _():
          pltpu.make_async_copy(buf, dst, wsem).wait()
          pltpu.async_copy(table_hbm.at[idx_v.at[j + NBUF]], buf, gsem)

      return carry

    lax.fori_loop(0, NCHUNK // NBUF, body, 0)

    for b in range(NBUF):
      pltpu.make_async_copy(
          bufs[b], out_hbm.at[pl.ds(base, CH)], wsem).wait()

  return emb


_emb = _make_kernel()


@jax.jit
def kernel(x, table):
  idx = x.reshape(NW, NCHUNK, CH).astype(jnp.int32)
  out = _emb(table, idx)
  return out.reshape(B, L, DIM)
